# Initial kernel scaffold; baseline (speedup 1.0000x reference)
import functools

import jax
import jax.numpy as jnp
from jax import lax
from jax.experimental import pallas as pl
from jax.experimental.pallas import tpu as pltpu
from jax.experimental.pallas import tpu_sc as plsc

N = 10000
E = 160000
D = 256
H = 3
C = 4

NC, NS = 2, 16
NW = NC * NS
NP = 10240
EP = 163840
EPW = EP // NW
BLK = 512
L = 16

_vmesh = plsc.VectorSubcoreMesh(
    core_axis_name="c", subcore_axis_name="s", num_cores=NC, num_subcores=NS
)


@functools.partial(
    pl.kernel,
    out_type=jax.ShapeDtypeStruct((NW, NP), jnp.float32),
    mesh=_vmesh,
    scratch_types=[
        pltpu.VMEM((EPW,), jnp.int32),
        pltpu.VMEM((NP,), jnp.float32),
    ],
)
def _sc_degree(dst_hbm, out_hbm, dst_v, deg_v):
    wid = lax.axis_index("c") * NS + lax.axis_index("s")
    pltpu.sync_copy(dst_hbm.at[pl.ds(wid * EPW, EPW)], dst_v)

    @pl.loop(0, NP, step=L)
    def _(i):
        deg_v[pl.ds(i, L)] = jnp.zeros((L,), jnp.float32)

    @pl.loop(0, EPW, step=L)
    def _(i):
        d = dst_v[pl.ds(i, L)]
        plsc.addupdate_scatter(deg_v, [d], jnp.ones((L,), jnp.float32))

    pltpu.sync_copy(deg_v, out_hbm.at[wid])


@functools.partial(
    pl.kernel,
    out_type=jax.ShapeDtypeStruct((NW, H, NP), jnp.float32),
    mesh=_vmesh,
    scratch_types=[
        pltpu.VMEM((EPW,), jnp.int32),
        pltpu.VMEM((EPW,), jnp.int32),
        pltpu.VMEM((NP,), jnp.float32),
        pltpu.VMEM((NP,), jnp.float32),
        pltpu.VMEM((NP,), jnp.float32),
        pltpu.VMEM((NP,), jnp.float32),
        pltpu.VMEM((NP,), jnp.float32),
        pltpu.VMEM((NP,), jnp.float32),
    ],
)
def _sc_aggregate(src_hbm, dst_hbm, g_hbm, out_hbm,
                  src_v, dst_v, g0, g1, g2, a0, a1, a2):
    wid = lax.axis_index("c") * NS + lax.axis_index("s")
    base = wid * EPW
    pltpu.sync_copy(src_hbm.at[pl.ds(base, EPW)], src_v)
    pltpu.sync_copy(dst_hbm.at[pl.ds(base, EPW)], dst_v)
    pltpu.sync_copy(g_hbm.at[0], g0)
    pltpu.sync_copy(g_hbm.at[1], g1)
    pltpu.sync_copy(g_hbm.at[2], g2)

    @pl.loop(0, NP, step=L)
    def _(i):
        z = jnp.zeros((L,), jnp.float32)
        a0[pl.ds(i, L)] = z
        a1[pl.ds(i, L)] = z
        a2[pl.ds(i, L)] = z

    @pl.loop(0, EPW, step=L)
    def _(i):
        s = src_v[pl.ds(i, L)]
        d = dst_v[pl.ds(i, L)]
        plsc.addupdate_scatter(a0, [d], plsc.load_gather(g0, [s]))
        plsc.addupdate_scatter(a1, [d], plsc.load_gather(g1, [s]))
        plsc.addupdate_scatter(a2, [d], plsc.load_gather(g2, [s]))

    pltpu.sync_copy(a0, out_hbm.at[wid, 0])
    pltpu.sync_copy(a1, out_hbm.at[wid, 1])
    pltpu.sync_copy(a2, out_hbm.at[wid, 2])


def _tc1_body(wt_ref, xt_ref, degp_ref, g_ref, dinv_ref):
    deg = jnp.sum(degp_ref[...], axis=0, keepdims=True) + 1.0
    dinv = lax.rsqrt(deg)
    h = jnp.dot(wt_ref[...], xt_ref[...],
                preferred_element_type=jnp.float32)
    g_ref[...] = h * dinv
    dinv_ref[...] = dinv


def _tc1(wt, xt, degp):
    return pl.pallas_call(
        _tc1_body,
        grid=(NP // BLK,),
        in_specs=[
            pl.BlockSpec((H, D), lambda i: (0, 0)),
            pl.BlockSpec((D, BLK), lambda i: (0, i)),
            pl.BlockSpec((NW, BLK), lambda i: (0, i)),
        ],
        out_specs=[
            pl.BlockSpec((H, BLK), lambda i: (0, i)),
            pl.BlockSpec((1, BLK), lambda i: (0, i)),
        ],
        out_shape=[
            jax.ShapeDtypeStruct((H, NP), jnp.float32),
            jax.ShapeDtypeStruct((1, NP), jnp.float32),
        ],
    )(wt, xt, degp)


def _tc2_body(accp_ref, g_ref, dinv_ref, bg_ref, wo_ref, bo_ref,
              hr_ref, z_ref):
    acc = jnp.sum(accp_ref[...], axis=0) + g_ref[...]
    hg = acc * dinv_ref[...] + bg_ref[...]
    hr = jnp.maximum(hg, 0.0)
    hr_ref[...] = hr
    z_ref[...] = jnp.dot(wo_ref[...], hr,
                         preferred_element_type=jnp.float32) + bo_ref[...]


def _tc2(accp, g, dinv, bg, wo, bo):
    return pl.pallas_call(
        _tc2_body,
        grid=(NP // BLK,),
        in_specs=[
            pl.BlockSpec((NW, H, BLK), lambda i: (0, 0, i)),
            pl.BlockSpec((H, BLK), lambda i: (0, i)),
            pl.BlockSpec((1, BLK), lambda i: (0, i)),
            pl.BlockSpec((H, 1), lambda i: (0, 0)),
            pl.BlockSpec((C, H), lambda i: (0, 0)),
            pl.BlockSpec((C, 1), lambda i: (0, 0)),
        ],
        out_specs=[
            pl.BlockSpec((H, BLK), lambda i: (0, i)),
            pl.BlockSpec((C, BLK), lambda i: (0, i)),
        ],
        out_shape=[
            jax.ShapeDtypeStruct((H, NP), jnp.float32),
            jax.ShapeDtypeStruct((C, NP), jnp.float32),
        ],
    )(accp, g, dinv, bg, wo, bo)


def kernel(x, edge_index, W_gcn, b_gcn, W_out, b_out):
    src = edge_index[0]
    dst = edge_index[1]
    npad = EP - E
    pad_idx = (N + (jnp.arange(npad, dtype=jnp.int32) % (NP - N))).astype(jnp.int32)
    src_p = jnp.concatenate([src, pad_idx])
    dst_p = jnp.concatenate([dst, pad_idx])

    xt = jnp.pad(x, ((0, NP - N), (0, 0))).T
    wt = W_gcn.T
    bg = b_gcn.reshape(H, 1)
    wo = W_out.T
    bo = b_out.reshape(C, 1)

    degp = _sc_degree(dst_p)
    g, dinv = _tc1(wt, xt, degp)
    accp = _sc_aggregate(src_p, dst_p, g)
    hrt, zt = _tc2(accp, g, dinv, bg, wo, bo)

    h_relu = hrt.T[:N]
    z = zt.T[:N]
    return (h_relu, z)


# speedup vs baseline: 35.5563x; 35.5563x over previous
import dataclasses
import functools

import jax
import jax.numpy as jnp
from jax import lax
from jax.experimental import pallas as pl
from jax.experimental.pallas import tpu as pltpu
from jax.experimental.pallas import tpu_sc as plsc

N = 10000
E = 160000
D = 256
H = 3
C = 4

NC, NS = 2, 16
NW = NC * NS
NP = 10240
EP = 163840
EPW = EP // NW
BLK = 512
L = 16

_vmesh = plsc.VectorSubcoreMesh(
    core_axis_name="c", subcore_axis_name="s", num_cores=NC, num_subcores=NS
)

_sc_params = pltpu.CompilerParams()
if "needs_layout_passes" in pltpu.CompilerParams.__dataclass_fields__:
    _sc_params = dataclasses.replace(_sc_params, needs_layout_passes=False)


@functools.partial(
    pl.kernel,
    out_type=jax.ShapeDtypeStruct((NW * NP,), jnp.float32),
    mesh=_vmesh,
    scratch_types=[
        pltpu.VMEM((EPW,), jnp.int32),
        pltpu.VMEM((NP,), jnp.float32),
    ],
    compiler_params=_sc_params,
)
def _sc_degree(dst_hbm, out_hbm, dst_v, deg_v):
    wid = lax.axis_index("c") * NS + lax.axis_index("s")
    pltpu.sync_copy(dst_hbm.at[pl.ds(wid * EPW, EPW)], dst_v)

    @pl.loop(0, NP, step=L)
    def _(i):
        deg_v[pl.ds(i, L)] = jnp.zeros((L,), jnp.float32)

    @pl.loop(0, EPW, step=L)
    def _(i):
        d = dst_v[pl.ds(i, L)]
        plsc.addupdate_scatter(deg_v, [d], jnp.ones((L,), jnp.float32))

    pltpu.sync_copy(deg_v, out_hbm.at[pl.ds(wid * NP, NP)])


@functools.partial(
    pl.kernel,
    out_type=jax.ShapeDtypeStruct((NW * H * NP,), jnp.float32),
    mesh=_vmesh,
    scratch_types=[
        pltpu.VMEM((EPW,), jnp.int32),
        pltpu.VMEM((EPW,), jnp.int32),
        pltpu.VMEM((NP,), jnp.float32),
        pltpu.VMEM((NP,), jnp.float32),
        pltpu.VMEM((NP,), jnp.float32),
        pltpu.VMEM((NP,), jnp.float32),
        pltpu.VMEM((NP,), jnp.float32),
        pltpu.VMEM((NP,), jnp.float32),
    ],
    compiler_params=_sc_params,
)
def _sc_aggregate(src_hbm, dst_hbm, g_hbm, out_hbm,
                  src_v, dst_v, g0, g1, g2, a0, a1, a2):
    wid = lax.axis_index("c") * NS + lax.axis_index("s")
    base = wid * EPW
    pltpu.sync_copy(src_hbm.at[pl.ds(base, EPW)], src_v)
    pltpu.sync_copy(dst_hbm.at[pl.ds(base, EPW)], dst_v)
    pltpu.sync_copy(g_hbm.at[pl.ds(0, NP)], g0)
    pltpu.sync_copy(g_hbm.at[pl.ds(NP, NP)], g1)
    pltpu.sync_copy(g_hbm.at[pl.ds(2 * NP, NP)], g2)

    @pl.loop(0, NP, step=L)
    def _(i):
        z = jnp.zeros((L,), jnp.float32)
        a0[pl.ds(i, L)] = z
        a1[pl.ds(i, L)] = z
        a2[pl.ds(i, L)] = z

    @pl.loop(0, EPW, step=L)
    def _(i):
        s = src_v[pl.ds(i, L)]
        d = dst_v[pl.ds(i, L)]
        plsc.addupdate_scatter(a0, [d], plsc.load_gather(g0, [s]))
        plsc.addupdate_scatter(a1, [d], plsc.load_gather(g1, [s]))
        plsc.addupdate_scatter(a2, [d], plsc.load_gather(g2, [s]))

    obase = wid * (H * NP)
    pltpu.sync_copy(a0, out_hbm.at[pl.ds(obase, NP)])
    pltpu.sync_copy(a1, out_hbm.at[pl.ds(obase + NP, NP)])
    pltpu.sync_copy(a2, out_hbm.at[pl.ds(obase + 2 * NP, NP)])


def _tc1_body(wt_ref, xt_ref, degp_ref, g_ref, dinv_ref):
    deg = jnp.sum(degp_ref[...], axis=0, keepdims=True) + 1.0
    dinv = lax.rsqrt(deg)
    h = jnp.dot(wt_ref[...], xt_ref[...],
                preferred_element_type=jnp.float32)
    g_ref[...] = h * dinv
    dinv_ref[...] = dinv


def _tc1(wt, xt, degp):
    return pl.pallas_call(
        _tc1_body,
        grid=(NP // BLK,),
        in_specs=[
            pl.BlockSpec((H, D), lambda i: (0, 0)),
            pl.BlockSpec((D, BLK), lambda i: (0, i)),
            pl.BlockSpec((NW, BLK), lambda i: (0, i)),
        ],
        out_specs=[
            pl.BlockSpec((H, BLK), lambda i: (0, i)),
            pl.BlockSpec((1, BLK), lambda i: (0, i)),
        ],
        out_shape=[
            jax.ShapeDtypeStruct((H, NP), jnp.float32),
            jax.ShapeDtypeStruct((1, NP), jnp.float32),
        ],
    )(wt, xt, degp)


def _tc2_body(accp_ref, g_ref, dinv_ref, bg_ref, wo_ref, bo_ref,
              hr_ref, z_ref):
    acc = jnp.sum(accp_ref[...], axis=0) + g_ref[...]
    hg = acc * dinv_ref[...] + bg_ref[...]
    hr = jnp.maximum(hg, 0.0)
    hr_ref[...] = hr
    z_ref[...] = jnp.dot(wo_ref[...], hr,
                         preferred_element_type=jnp.float32) + bo_ref[...]


def _tc2(accp, g, dinv, bg, wo, bo):
    return pl.pallas_call(
        _tc2_body,
        grid=(NP // BLK,),
        in_specs=[
            pl.BlockSpec((NW, H, BLK), lambda i: (0, 0, i)),
            pl.BlockSpec((H, BLK), lambda i: (0, i)),
            pl.BlockSpec((1, BLK), lambda i: (0, i)),
            pl.BlockSpec((H, 1), lambda i: (0, 0)),
            pl.BlockSpec((C, H), lambda i: (0, 0)),
            pl.BlockSpec((C, 1), lambda i: (0, 0)),
        ],
        out_specs=[
            pl.BlockSpec((H, BLK), lambda i: (0, i)),
            pl.BlockSpec((C, BLK), lambda i: (0, i)),
        ],
        out_shape=[
            jax.ShapeDtypeStruct((H, NP), jnp.float32),
            jax.ShapeDtypeStruct((C, NP), jnp.float32),
        ],
    )(accp, g, dinv, bg, wo, bo)


def kernel(x, edge_index, W_gcn, b_gcn, W_out, b_out):
    src = edge_index[0]
    dst = edge_index[1]
    npad = EP - E
    pad_idx = (N + (jnp.arange(npad, dtype=jnp.int32) % (NP - N))).astype(jnp.int32)
    src_p = jnp.concatenate([src, pad_idx])
    dst_p = jnp.concatenate([dst, pad_idx])

    xt = jnp.pad(x, ((0, NP - N), (0, 0))).T
    wt = W_gcn.T
    bg = b_gcn.reshape(H, 1)
    wo = W_out.T
    bo = b_out.reshape(C, 1)

    degp = _sc_degree(dst_p).reshape(NW, NP)
    g, dinv = _tc1(wt, xt, degp)
    accp = _sc_aggregate(src_p, dst_p, g.reshape(H * NP))
    hrt, zt = _tc2(accp.reshape(NW, H, NP), g, dinv, bg, wo, bo)

    h_relu = hrt.T[:N]
    z = zt.T[:N]
    return (h_relu, z)
